# baseline (speedup 1.0000x reference)
import functools
import math

import jax
import jax.numpy as jnp
from jax import lax
from jax.experimental import pallas as pl
from jax.experimental.pallas import tpu as pltpu
from jax.experimental.pallas import tpu_sc as plsc

VOCAB = 1000000
EMB = 64
SCALE = math.sqrt(EMB)
BLK = 128


def _make_sc_embed(batch: int, seq: int):
    info = plsc.get_sparse_core_info()
    nc, ns = info.num_cores, info.num_subcores
    nw = nc * ns
    assert batch % (BLK * nw) == 0 or batch == BLK * nw
    mesh = plsc.VectorSubcoreMesh(core_axis_name="c", subcore_axis_name="s")

    @functools.partial(
        pl.kernel,
        out_type=jax.ShapeDtypeStruct((seq, EMB, batch), jnp.float32),
        mesh=mesh,
        scratch_types=[
            pltpu.VMEM((seq, BLK), jnp.int32),
            pltpu.VMEM((seq, BLK), jnp.int32),
            pltpu.VMEM((BLK, 128), jnp.float32),
            pltpu.VMEM((EMB, BLK), jnp.float32),
            pltpu.SemaphoreType.DMA,
        ],
        compiler_params=pltpu.CompilerParams(needs_layout_passes=False),
    )
    def sc_embed(tok_hbm, tab_hbm, out_hbm, idx_v, idxg_v, rows_v, tr_v, sem):
        wid = lax.axis_index("s") * nc + lax.axis_index("c")
        b0 = wid * BLK
        pltpu.sync_copy(tok_hbm.at[:, pl.ds(b0, BLK)], idx_v)

        iota16 = lax.iota(jnp.int32, 16)

        def idx_body(k, _):
            s = k // (BLK // 16)
            i0 = (k % (BLK // 16)) * 16
            tv = idx_v[s, pl.ds(i0, 16)]
            idxg_v[s, pl.ds(i0, 16)] = lax.shift_right_logical(tv, 1)
            return 0

        lax.fori_loop(0, seq * (BLK // 16), idx_body, 0)

        def cell_body(s, _):
            pltpu.async_copy(tab_hbm.at[idxg_v.at[s]], rows_v, sem).wait()

            for j in range(BLK // 16):
                i0 = j * 16
                tv = idx_v[s, pl.ds(i0, 16)]
                ev0 = lax.mul(lax.bitwise_and(tv, 1), 64)
                iv = lax.add(lax.broadcast(i0, (16,)), iota16)

                def e_body(e, col):
                    vals = plsc.load_gather(rows_v, [iv, col])
                    tr_v[e, pl.ds(i0, 16)] = vals * SCALE
                    return lax.add(col, lax.broadcast(1, (16,)))

                lax.fori_loop(0, EMB, e_body, ev0, unroll=4)

            pltpu.sync_copy(tr_v, out_hbm.at[s, :, pl.ds(b0, BLK)])
            return 0

        lax.fori_loop(0, seq, cell_body, 0)

    return sc_embed


@jax.jit
def kernel(tokens, table):
    batch, seq = tokens.shape
    tok_t = jnp.transpose(tokens.astype(jnp.int32))
    tab2 = jnp.reshape(table, (VOCAB // 2, 2 * EMB))
    out3d = _make_sc_embed(batch, seq)(tok_t, tab2)
    return jnp.transpose(out3d, (2, 0, 1))


# speedup vs baseline: 1.1578x; 1.1578x over previous
import functools
import math

import jax
import jax.numpy as jnp
from jax import lax
from jax.experimental import pallas as pl
from jax.experimental.pallas import tpu as pltpu
from jax.experimental.pallas import tpu_sc as plsc

VOCAB = 1000000
EMB = 64
SCALE = math.sqrt(EMB)
BLK = 128


def _make_sc_embed(batch: int, seq: int):
    info = plsc.get_sparse_core_info()
    nc, ns = info.num_cores, info.num_subcores
    nw = nc * ns
    assert batch == BLK * nw and seq % 2 == 0
    mesh = plsc.VectorSubcoreMesh(core_axis_name="c", subcore_axis_name="s")

    @functools.partial(
        pl.kernel,
        out_type=jax.ShapeDtypeStruct((seq, EMB, batch), jnp.float32),
        mesh=mesh,
        scratch_types=[
            pltpu.VMEM((seq, BLK), jnp.int32),
            pltpu.VMEM((2, BLK), jnp.int32),
            pltpu.VMEM((2, BLK, 128), jnp.float32),
            pltpu.VMEM((2, EMB, BLK), jnp.float32),
            pltpu.SemaphoreType.DMA,
            pltpu.SemaphoreType.DMA,
            pltpu.SemaphoreType.DMA,
            pltpu.SemaphoreType.DMA,
        ],
        compiler_params=pltpu.CompilerParams(needs_layout_passes=False),
    )
    def sc_embed(tok_hbm, tab_hbm, out_hbm, idx_v, idxg_v, rows_v, tr_v,
                 gsem0, gsem1, osem0, osem1):
        gsem = (gsem0, gsem1)
        osem = (osem0, osem1)
        wid = lax.axis_index("s") * nc + lax.axis_index("c")
        b0 = wid * BLK
        pltpu.sync_copy(tok_hbm.at[:, pl.ds(b0, BLK)], idx_v)

        iota16 = lax.iota(jnp.int32, 16)

        def fill_idxg(s, slot):
            for j in range(BLK // 16):
                i0 = j * 16
                tv = idx_v[s, pl.ds(i0, 16)]
                idxg_v[slot, pl.ds(i0, 16)] = lax.shift_right_logical(tv, 1)

        def start_gather(s, b):
            pltpu.async_copy(tab_hbm.at[idxg_v.at[b]], rows_v.at[b], gsem[b])

        fill_idxg(0, 0)
        start_gather(0, 0)
        fill_idxg(1, 1)
        start_gather(1, 1)

        def pair_body(p, _):
            for b in range(2):
                s = 2 * p + b

                @pl.when(p > 0)
                def _():
                    pltpu.make_async_copy(
                        tr_v.at[b], out_hbm.at[s, :, pl.ds(b0, BLK)], osem[b]
                    ).wait()

                pltpu.make_async_copy(
                    tab_hbm.at[idxg_v.at[b]], rows_v.at[b], gsem[b]
                ).wait()

                for j in range(BLK // 16):
                    i0 = j * 16
                    tv = idx_v[s, pl.ds(i0, 16)]
                    ev0 = lax.mul(lax.bitwise_and(tv, 1), 64)
                    iv = lax.add(lax.broadcast(i0, (16,)), iota16)

                    def e_body(e, _):
                        col = lax.add(ev0, lax.broadcast(e, (16,)))
                        vals = plsc.load_gather(rows_v.at[b], [iv, col])
                        tr_v[b, e, pl.ds(i0, 16)] = vals * SCALE
                        return 0

                    lax.fori_loop(0, EMB, e_body, 0, unroll=16)

                @pl.when(s < seq - 2)
                def _():
                    fill_idxg(s + 2, b)
                    pltpu.async_copy(
                        tab_hbm.at[idxg_v.at[b]], rows_v.at[b], gsem[b]
                    )

                pltpu.async_copy(
                    tr_v.at[b], out_hbm.at[s, :, pl.ds(b0, BLK)], osem[b]
                )
            return 0

        lax.fori_loop(0, seq // 2, pair_body, 0)

        for b in range(2):
            s = seq - 2 + b
            pltpu.make_async_copy(
                tr_v.at[b], out_hbm.at[s, :, pl.ds(b0, BLK)], osem[b]
            ).wait()

    return sc_embed


@jax.jit
def kernel(tokens, table):
    batch, seq = tokens.shape
    tok_t = jnp.transpose(tokens.astype(jnp.int32))
    tab2 = jnp.reshape(table, (VOCAB // 2, 2 * EMB))
    out3d = _make_sc_embed(batch, seq)(tok_t, tab2)
    return jnp.transpose(out3d, (2, 0, 1))


# speedup vs baseline: 1.8123x; 1.5653x over previous
import functools
import math

import jax
import jax.numpy as jnp
from jax import lax
from jax.experimental import pallas as pl
from jax.experimental.pallas import tpu as pltpu
from jax.experimental.pallas import tpu_sc as plsc

VOCAB = 1000000
EMB = 64
SCALE = math.sqrt(EMB)
BLK = 128


def _make_sc_embed(batch: int, seq: int):
    info = plsc.get_sparse_core_info()
    nc, ns = info.num_cores, info.num_subcores
    nw = nc * ns
    assert batch == BLK * nw and seq % 2 == 0
    mesh = plsc.VectorSubcoreMesh(core_axis_name="c", subcore_axis_name="s")

    @functools.partial(
        pl.kernel,
        out_type=jax.ShapeDtypeStruct((seq, EMB, batch), jnp.float32),
        mesh=mesh,
        scratch_types=[
            pltpu.VMEM((seq, BLK), jnp.int32),
            pltpu.VMEM((2, BLK), jnp.int32),
            pltpu.VMEM((2, BLK, 128), jnp.float32),
            pltpu.VMEM((2, EMB, BLK), jnp.float32),
            pltpu.SemaphoreType.DMA,
            pltpu.SemaphoreType.DMA,
            pltpu.SemaphoreType.DMA,
            pltpu.SemaphoreType.DMA,
        ],
        compiler_params=pltpu.CompilerParams(needs_layout_passes=False),
    )
    def sc_embed(tok_hbm, tab_hbm, out_hbm, idx_v, idxg_v, rows_v, tr_v,
                 gsem0, gsem1, osem0, osem1):
        gsem = (gsem0, gsem1)
        osem = (osem0, osem1)
        wid = lax.axis_index("s") * nc + lax.axis_index("c")
        b0 = wid * BLK
        pltpu.sync_copy(tok_hbm.at[:, pl.ds(b0, BLK)], idx_v)

        iota16 = lax.iota(jnp.int32, 16)

        def fill_idxg(s, slot):
            for j in range(BLK // 16):
                i0 = j * 16
                tv = idx_v[s, pl.ds(i0, 16)]
                idxg_v[slot, pl.ds(i0, 16)] = lax.shift_right_logical(tv, 1)

        def start_gather(s, b):
            pltpu.async_copy(tab_hbm.at[idxg_v.at[b]], rows_v.at[b], gsem[b])

        fill_idxg(0, 0)
        start_gather(0, 0)
        fill_idxg(1, 1)
        start_gather(1, 1)

        def pair_body(p, _):
            for b in range(2):
                s = 2 * p + b

                @pl.when(p > 0)
                def _():
                    pltpu.make_async_copy(
                        tr_v.at[b], out_hbm.at[s, :, pl.ds(b0, BLK)], osem[b]
                    ).wait()

                pltpu.make_async_copy(
                    tab_hbm.at[idxg_v.at[b]], rows_v.at[b], gsem[b]
                ).wait()

                for j in range(BLK // 16):
                    i0 = j * 16
                    tv = idx_v[s, pl.ds(i0, 16)]
                    ev0 = lax.mul(lax.bitwise_and(tv, 1), 64)
                    iv = lax.add(lax.broadcast(i0, (16,)), iota16)

                    @plsc.parallel_loop(0, EMB, 1, unroll=16)
                    def e_body(e):
                        col = lax.add(ev0, lax.broadcast(e, (16,)))
                        vals = plsc.load_gather(rows_v.at[b], [iv, col])
                        tr_v[b, e, pl.ds(i0, 16)] = vals * SCALE

                @pl.when(s < seq - 2)
                def _():
                    fill_idxg(s + 2, b)
                    pltpu.async_copy(
                        tab_hbm.at[idxg_v.at[b]], rows_v.at[b], gsem[b]
                    )

                pltpu.async_copy(
                    tr_v.at[b], out_hbm.at[s, :, pl.ds(b0, BLK)], osem[b]
                )
            return 0

        lax.fori_loop(0, seq // 2, pair_body, 0)

        for b in range(2):
            s = seq - 2 + b
            pltpu.make_async_copy(
                tr_v.at[b], out_hbm.at[s, :, pl.ds(b0, BLK)], osem[b]
            ).wait()

    return sc_embed


@jax.jit
def kernel(tokens, table):
    batch, seq = tokens.shape
    tok_t = jnp.transpose(tokens.astype(jnp.int32))
    tab2 = jnp.reshape(table, (VOCAB // 2, 2 * EMB))
    out3d = _make_sc_embed(batch, seq)(tok_t, tab2)
    return jnp.transpose(out3d, (2, 0, 1))


# speedup vs baseline: 1.8348x; 1.0124x over previous
import functools
import math

import jax
import jax.numpy as jnp
from jax import lax
from jax.experimental import pallas as pl
from jax.experimental.pallas import tpu as pltpu
from jax.experimental.pallas import tpu_sc as plsc

VOCAB = 1000000
EMB = 64
SCALE = math.sqrt(EMB)
BLK = 128


def _make_sc_embed(batch: int, seq: int):
    info = plsc.get_sparse_core_info()
    nc, ns = info.num_cores, info.num_subcores
    nw = nc * ns
    assert batch == BLK * nw and seq % 2 == 0
    mesh = plsc.VectorSubcoreMesh(core_axis_name="c", subcore_axis_name="s")

    @functools.partial(
        pl.kernel,
        out_type=jax.ShapeDtypeStruct((seq, EMB, batch), jnp.float32),
        mesh=mesh,
        scratch_types=[
            pltpu.VMEM((seq, BLK), jnp.int32),
            pltpu.VMEM((2, BLK), jnp.int32),
            pltpu.VMEM((2, BLK, 128), jnp.float32),
            pltpu.VMEM((2, EMB, BLK), jnp.float32),
            pltpu.SemaphoreType.DMA,
            pltpu.SemaphoreType.DMA,
            pltpu.SemaphoreType.DMA,
            pltpu.SemaphoreType.DMA,
        ],
        compiler_params=pltpu.CompilerParams(needs_layout_passes=False),
    )
    def sc_embed(tok_hbm, tab_hbm, out_hbm, idx_v, idxg_v, rows_v, tr_v,
                 gsem0, gsem1, osem0, osem1):
        gsem = (gsem0, gsem1)
        osem = (osem0, osem1)
        wid = lax.axis_index("s") * nc + lax.axis_index("c")
        b0 = wid * BLK
        pltpu.sync_copy(tok_hbm.at[:, pl.ds(b0, BLK)], idx_v)

        iota16 = lax.iota(jnp.int32, 16)

        def fill_idxg(s, slot):
            for j in range(BLK // 16):
                i0 = j * 16
                tv = idx_v[s, pl.ds(i0, 16)]
                idxg_v[slot, pl.ds(i0, 16)] = lax.shift_right_logical(tv, 1)

        def start_gather(s, b):
            pltpu.async_copy(tab_hbm.at[idxg_v.at[b]], rows_v.at[b], gsem[b])

        fill_idxg(0, 0)
        start_gather(0, 0)
        fill_idxg(1, 1)
        start_gather(1, 1)

        def pair_body(p, _):
            for b in range(2):
                s = 2 * p + b

                @pl.when(p > 0)
                def _():
                    pltpu.make_async_copy(
                        tr_v.at[b], out_hbm.at[s, :, pl.ds(b0, BLK)], osem[b]
                    ).wait()

                pltpu.make_async_copy(
                    tab_hbm.at[idxg_v.at[b]], rows_v.at[b], gsem[b]
                ).wait()

                tvs = [idx_v[s, pl.ds(j * 16, 16)] for j in range(BLK // 16)]
                ev0s = [lax.mul(lax.bitwise_and(tv, 1), 64) for tv in tvs]
                ivs = [lax.add(lax.broadcast(j * 16, (16,)), iota16)
                       for j in range(BLK // 16)]

                @plsc.parallel_loop(0, EMB, 1, unroll=4)
                def e_body(e):
                    eb = lax.broadcast(e, (16,))
                    for j in range(BLK // 16):
                        col = lax.add(ev0s[j], eb)
                        vals = plsc.load_gather(rows_v.at[b], [ivs[j], col])
                        tr_v[b, e, pl.ds(j * 16, 16)] = vals * SCALE

                @pl.when(s < seq - 2)
                def _():
                    fill_idxg(s + 2, b)
                    pltpu.async_copy(
                        tab_hbm.at[idxg_v.at[b]], rows_v.at[b], gsem[b]
                    )

                pltpu.async_copy(
                    tr_v.at[b], out_hbm.at[s, :, pl.ds(b0, BLK)], osem[b]
                )
            return 0

        lax.fori_loop(0, seq // 2, pair_body, 0)

        for b in range(2):
            s = seq - 2 + b
            pltpu.make_async_copy(
                tr_v.at[b], out_hbm.at[s, :, pl.ds(b0, BLK)], osem[b]
            ).wait()

    return sc_embed


@jax.jit
def kernel(tokens, table):
    batch, seq = tokens.shape
    tok_t = jnp.transpose(tokens.astype(jnp.int32))
    tab2 = jnp.reshape(table, (VOCAB // 2, 2 * EMB))
    out3d = _make_sc_embed(batch, seq)(tok_t, tab2)
    return jnp.transpose(out3d, (2, 0, 1))
